# baseline (speedup 1.0000x reference)
import jax
import jax.numpy as jnp
from jax import lax
from jax.experimental import pallas as pl
from jax.experimental.pallas import tpu as pltpu
from jax.experimental.pallas import tpu_sc as plsc

_V = 1000000
_D = 64
_B = 16384
_NNEG = 20
_NC = 2
_NS = 16
_NW = _NC * _NS
_BPW = _B // _NW
_CH = 32
_NCHUNK = _BPW // _CH
_SCORE_COLS = 32


_NR = _CH * (_NNEG + 2)
_NG = _NR // 16


_W = _D // 2


def _sc_body(center_hbm, context_hbm, negf_hbm, in_emb_hbm, out_emb_hbm,
             scores_hbm, idx_v0, idx_v1, rows_v0, rows_v1, sc_v, pacc,
             sem0, sem1):
    wid = lax.axis_index("s") * _NC + lax.axis_index("c")
    iota = lax.iota(jnp.int32, 16)
    iota_hi = iota + 16
    sems = (sem0, sem1)
    idx_bufs = (idx_v0, idx_v1)
    rows_bufs = (rows_v0, rows_v1)
    himask = jnp.full((16,), -65536, jnp.int32)

    def unpack2(w):
        ev = plsc.bitcast(jnp.left_shift(w, 16), jnp.float32)
        od = plsc.bitcast(jnp.bitwise_and(w, himask), jnp.float32)
        return ev, od

    def stage(ch, buf, sem):
        idx_v = idx_bufs[buf]
        rows_v = rows_bufs[buf]
        base = wid * _BPW + ch * _CH
        pltpu.sync_copy(center_hbm.at[pl.ds(base, _CH)],
                        idx_v.at[pl.ds(0, _CH)])
        pltpu.sync_copy(context_hbm.at[pl.ds(base, _CH)],
                        idx_v.at[pl.ds(_CH, _CH)])
        pltpu.sync_copy(negf_hbm.at[pl.ds(base * _NNEG, _CH * _NNEG)],
                        idx_v.at[pl.ds(2 * _CH, _CH * _NNEG)])

        def issue_c(g, c2):
            v = idx_v[pl.ds(g * 16, 16)]
            for k in range(16):
                pltpu.async_copy(in_emb_hbm.at[pl.ds(v[k] * _W, _W)],
                                 rows_v.at[pl.ds((g * 16 + k) * _W, _W)], sem)
            return c2

        lax.fori_loop(0, _CH // 16, issue_c, None)

        def issue_o(g, c2):
            v = idx_v[pl.ds(g * 16, 16)]
            for k in range(16):
                pltpu.async_copy(out_emb_hbm.at[pl.ds(v[k] * _W, _W)],
                                 rows_v.at[pl.ds((g * 16 + k) * _W, _W)], sem)
            return c2

        lax.fori_loop(_CH // 16, _NG, issue_o, None)

    def consume(ch, buf):
        rows_v = rows_bufs[buf]
        pltpu.make_async_copy(out_emb_hbm.at[pl.ds(0, _NR * _W)], rows_v,
                              sems[buf]).wait()

        def b_body(b, carry2):
            ce0, co0 = unpack2(rows_v[pl.ds(b * _W, 16)])
            ce1, co1 = unpack2(rows_v[pl.ds(b * _W + 16, 16)])
            for j in range(_NNEG + 1):
                if j == 0:
                    row = _CH + b
                else:
                    row = 2 * _CH + b * _NNEG + (j - 1)
                re0, ro0 = unpack2(rows_v[pl.ds(row * _W, 16)])
                re1, ro1 = unpack2(rows_v[pl.ds(row * _W + 16, 16)])
                pacc[j, :] = (ce0 * re0 + co0 * ro0 + ce1 * re1 + co1 * ro1)
            s_lo = jnp.zeros((16,), jnp.float32)
            s_hi = jnp.zeros((16,), jnp.float32)
            for l in range(16):
                dcol = (iota + l) & 15
                s_lo = s_lo + plsc.load_gather(pacc, [iota, dcol])
                s_hi = s_hi + plsc.load_gather(pacc, [iota_hi, dcol])
            sc_v[b, pl.ds(0, 16)] = s_lo
            sc_v[b, pl.ds(16, 16)] = s_hi
            return carry2

        lax.fori_loop(0, _CH, b_body, None)
        base = wid * _BPW + ch * _CH
        pltpu.sync_copy(sc_v, scores_hbm.at[pl.ds(base, _CH), :])

    stage(0, 0, sem0)

    def pair_body(i, carry):
        ch = 2 * i
        stage(ch + 1, 1, sem1)
        consume(ch, 0)

        @pl.when(ch + 2 < _NCHUNK)
        def _():
            stage(ch + 2, 0, sem0)

        consume(ch + 1, 1)
        return carry

    lax.fori_loop(0, _NCHUNK // 2, pair_body, None)


_sc_scores = pl.kernel(
    _sc_body,
    out_type=jax.ShapeDtypeStruct((_B, _SCORE_COLS), jnp.float32),
    mesh=plsc.VectorSubcoreMesh(core_axis_name="c", subcore_axis_name="s",
                                num_cores=_NC, num_subcores=_NS),
    compiler_params=pltpu.CompilerParams(needs_layout_passes=False),
    scratch_types=[
        pltpu.VMEM((_NR,), jnp.int32),
        pltpu.VMEM((_NR,), jnp.int32),
        pltpu.VMEM((_NR * _W,), jnp.int32),
        pltpu.VMEM((_NR * _W,), jnp.int32),
        pltpu.VMEM((_CH, _SCORE_COLS), jnp.float32),
        pltpu.VMEM((2 * _CH, 16), jnp.float32),
        pltpu.SemaphoreType.DMA,
        pltpu.SemaphoreType.DMA,
    ],
)


def _loss_body(s_ref, o_ref):
    s = s_ref[...]
    col = lax.broadcasted_iota(jnp.int32, s.shape, 1)
    x = jnp.where(col == 0, s, -s)
    ls = jnp.minimum(x, 0.0) - jnp.log(1.0 + jnp.exp(-jnp.abs(x)))
    ls = jnp.where(col < _NNEG + 1, ls, 0.0)
    o_ref[...] = (-jnp.sum(ls) / _B).reshape(1, 1)


_loss = pl.pallas_call(
    _loss_body,
    out_shape=jax.ShapeDtypeStruct((1, 1), jnp.float32),
)


def _pack_bf16_words(table):
    i = lax.bitcast_convert_type(table, jnp.int32)
    r = i + 0x7FFF + (lax.shift_right_logical(i, 16) & 1)
    r_e = r[:, 0::2]
    r_o = r[:, 1::2]
    w = (lax.shift_right_logical(r_e, 16) | (r_o & jnp.int32(-65536)))
    return w.reshape(_V * _W)


def kernel(center, context, negatives, input_emb, output_emb):
    negf = negatives.reshape(_B * _NNEG)
    in_w = _pack_bf16_words(input_emb)
    out_w = _pack_bf16_words(output_emb)
    scores = _sc_scores(center, context, negf, in_w, out_w)
    return _loss(scores)[0, 0]


# speedup vs baseline: 4.5353x; 4.5353x over previous
import jax
import jax.numpy as jnp
from jax import lax
from jax.experimental import pallas as pl
from jax.experimental.pallas import tpu as pltpu
from jax.experimental.pallas import tpu_sc as plsc

_V = 1000000
_D = 64
_B = 16384
_NNEG = 20
_NC = 2
_NS = 16
_NW = _NC * _NS
_BPW = _B // _NW
_CH = 16
_NCHUNK = _BPW // _CH
_SCORE_COLS = 32

_NR = _CH * (_NNEG + 2)
_NG = _NR // 16


def _sc_body(idxall_hbm, in_emb_hbm, out_emb_hbm,
             scores_hbm, idx_v0, idx_v1, rows_v0, rows_v1, sc_v, pacc,
             sem0, sem1):
    wid = lax.axis_index("s") * _NC + lax.axis_index("c")
    iota = lax.iota(jnp.int32, 16)
    iota_hi = iota + 16
    sems = (sem0, sem1)
    idx_bufs = (idx_v0, idx_v1)
    rows_bufs = (rows_v0, rows_v1)

    def stage(ch, buf, sem):
        idx_v = idx_bufs[buf]
        rows_v = rows_bufs[buf]
        pltpu.sync_copy(
            idxall_hbm.at[pl.ds((wid * _NCHUNK + ch) * _NR, _NR)], idx_v)

        def issue_c(g, c2):
            v = idx_v[pl.ds(g * 16, 16)]
            for k in range(16):
                pltpu.async_copy(in_emb_hbm.at[v[k]],
                                 rows_v.at[g * 16 + k], sem)
            return c2

        lax.fori_loop(0, _CH // 16, issue_c, None)

        def issue_o(g, c2):
            v = idx_v[pl.ds(g * 16, 16)]
            for k in range(16):
                pltpu.async_copy(out_emb_hbm.at[v[k]],
                                 rows_v.at[g * 16 + k], sem)
            return c2

        lax.fori_loop(_CH // 16, _NG, issue_o, None)

    def consume(ch, buf):
        rows_v = rows_bufs[buf]
        pltpu.make_async_copy(out_emb_hbm.at[pl.ds(0, _NR)], rows_v,
                              sems[buf]).wait()

        def b_body(b, carry2):
            c0 = rows_v[b, pl.ds(0, 16)]
            c1 = rows_v[b, pl.ds(16, 16)]
            c2 = rows_v[b, pl.ds(32, 16)]
            c3 = rows_v[b, pl.ds(48, 16)]
            for j in range(_NNEG + 1):
                if j == 0:
                    row = _CH + b
                else:
                    row = 2 * _CH + b * _NNEG + (j - 1)
                pacc[j, :] = (c0 * rows_v[row, pl.ds(0, 16)]
                              + c1 * rows_v[row, pl.ds(16, 16)]
                              + c2 * rows_v[row, pl.ds(32, 16)]
                              + c3 * rows_v[row, pl.ds(48, 16)])
            s_lo = jnp.zeros((16,), jnp.float32)
            s_hi = jnp.zeros((16,), jnp.float32)
            for l in range(16):
                dcol = (iota + l) & 15
                s_lo = s_lo + plsc.load_gather(pacc, [iota, dcol])
                s_hi = s_hi + plsc.load_gather(pacc, [iota_hi, dcol])
            sc_v[b, pl.ds(0, 16)] = s_lo
            sc_v[b, pl.ds(16, 16)] = s_hi
            return carry2

        lax.fori_loop(0, _CH, b_body, None)
        base = wid * _BPW + ch * _CH
        pltpu.sync_copy(sc_v, scores_hbm.at[pl.ds(base, _CH), :])

    stage(0, 0, sem0)

    def pair_body(i, carry):
        ch = 2 * i
        stage(ch + 1, 1, sem1)
        consume(ch, 0)

        @pl.when(ch + 2 < _NCHUNK)
        def _():
            stage(ch + 2, 0, sem0)

        consume(ch + 1, 1)
        return carry

    lax.fori_loop(0, _NCHUNK // 2, pair_body, None)


_sc_scores = pl.kernel(
    _sc_body,
    out_type=jax.ShapeDtypeStruct((_B, _SCORE_COLS), jnp.float32),
    mesh=plsc.VectorSubcoreMesh(core_axis_name="c", subcore_axis_name="s",
                                num_cores=_NC, num_subcores=_NS),
    compiler_params=pltpu.CompilerParams(needs_layout_passes=False),
    scratch_types=[
        pltpu.VMEM((_NR,), jnp.int32),
        pltpu.VMEM((_NR,), jnp.int32),
        pltpu.VMEM((_NR, _D), jnp.float32),
        pltpu.VMEM((_NR, _D), jnp.float32),
        pltpu.VMEM((_CH, _SCORE_COLS), jnp.float32),
        pltpu.VMEM((2 * _CH, 16), jnp.float32),
        pltpu.SemaphoreType.DMA,
        pltpu.SemaphoreType.DMA,
    ],
)


def _loss_body(s_ref, o_ref):
    s = s_ref[...]
    col = lax.broadcasted_iota(jnp.int32, s.shape, 1)
    x = jnp.where(col == 0, s, -s)
    ls = jnp.minimum(x, 0.0) - jnp.log(1.0 + jnp.exp(-jnp.abs(x)))
    ls = jnp.where(col < _NNEG + 1, ls, 0.0)
    o_ref[...] = (-jnp.sum(ls) / _B).reshape(1, 1)


_loss = pl.pallas_call(
    _loss_body,
    out_shape=jax.ShapeDtypeStruct((1, 1), jnp.float32),
)


def kernel(center, context, negatives, input_emb, output_emb):
    idxall = jnp.concatenate(
        [center.reshape(-1, _CH), context.reshape(-1, _CH),
         negatives.reshape(-1, _CH * _NNEG)], axis=1).reshape(-1)
    scores = _sc_scores(idxall, input_emb, output_emb)
    return _loss(scores)[0, 0]


# speedup vs baseline: 4.6325x; 1.0214x over previous
import jax
import jax.numpy as jnp
from jax import lax
from jax.experimental import pallas as pl
from jax.experimental.pallas import tpu as pltpu
from jax.experimental.pallas import tpu_sc as plsc

_V = 1000000
_D = 64
_B = 16384
_NNEG = 20
_NC = 2
_NS = 16
_NW = _NC * _NS
_BPW = _B // _NW
_CH = 16
_NCHUNK = _BPW // _CH
_SCORE_COLS = 32

_NR = _CH * (_NNEG + 2)
_NG = _NR // 16


def _sc_body(idxall_hbm, in_emb_hbm, out_emb_hbm,
             scores_hbm, idx_v, rows_v0, rows_v1, sc_v, pacc,
             sem0, sem1):
    wid = lax.axis_index("s") * _NC + lax.axis_index("c")
    iota = lax.iota(jnp.int32, 16)
    iota_hi = iota + 16
    sems = (sem0, sem1)
    rows_bufs = (rows_v0, rows_v1)

    pltpu.sync_copy(
        idxall_hbm.at[pl.ds(wid * _NCHUNK * _NR, _NCHUNK * _NR)], idx_v)

    def stage(ch, buf, sem):
        rows_v = rows_bufs[buf]

        def issue_c(g, c2):
            v = idx_v[pl.ds(ch * _NR + g * 16, 16)]
            for k in range(16):
                pltpu.async_copy(in_emb_hbm.at[v[k]],
                                 rows_v.at[g * 16 + k], sem)
            return c2

        lax.fori_loop(0, _CH // 16, issue_c, None)

        def issue_o(g, c2):
            v = idx_v[pl.ds(ch * _NR + g * 16, 16)]
            for k in range(16):
                pltpu.async_copy(out_emb_hbm.at[v[k]],
                                 rows_v.at[g * 16 + k], sem)
            return c2

        lax.fori_loop(_CH // 16, _NG, issue_o, None)

    def consume(ch, buf):
        rows_v = rows_bufs[buf]
        pltpu.make_async_copy(out_emb_hbm.at[pl.ds(0, _NR)], rows_v,
                              sems[buf]).wait()

        def b_body(b, carry2):
            c0 = rows_v[b, pl.ds(0, 16)]
            c1 = rows_v[b, pl.ds(16, 16)]
            c2 = rows_v[b, pl.ds(32, 16)]
            c3 = rows_v[b, pl.ds(48, 16)]
            for j in range(_NNEG + 1):
                if j == 0:
                    row = _CH + b
                else:
                    row = 2 * _CH + b * _NNEG + (j - 1)
                pacc[j, :] = (c0 * rows_v[row, pl.ds(0, 16)]
                              + c1 * rows_v[row, pl.ds(16, 16)]
                              + c2 * rows_v[row, pl.ds(32, 16)]
                              + c3 * rows_v[row, pl.ds(48, 16)])
            s_lo = jnp.zeros((16,), jnp.float32)
            s_hi = jnp.zeros((16,), jnp.float32)
            for l in range(16):
                dcol = (iota + l) & 15
                s_lo = s_lo + plsc.load_gather(pacc, [iota, dcol])
                s_hi = s_hi + plsc.load_gather(pacc, [iota_hi, dcol])
            sc_v[b, pl.ds(0, 16)] = s_lo
            sc_v[b, pl.ds(16, 16)] = s_hi
            return carry2

        lax.fori_loop(0, _CH, b_body, None)
        base = wid * _BPW + ch * _CH
        pltpu.sync_copy(sc_v, scores_hbm.at[pl.ds(base, _CH), :])

    stage(0, 0, sem0)

    def pair_body(i, carry):
        ch = 2 * i
        stage(ch + 1, 1, sem1)
        consume(ch, 0)

        @pl.when(ch + 2 < _NCHUNK)
        def _():
            stage(ch + 2, 0, sem0)

        consume(ch + 1, 1)
        return carry

    lax.fori_loop(0, _NCHUNK // 2, pair_body, None)


_sc_scores = pl.kernel(
    _sc_body,
    out_type=jax.ShapeDtypeStruct((_B, _SCORE_COLS), jnp.float32),
    mesh=plsc.VectorSubcoreMesh(core_axis_name="c", subcore_axis_name="s",
                                num_cores=_NC, num_subcores=_NS),
    compiler_params=pltpu.CompilerParams(needs_layout_passes=False),
    scratch_types=[
        pltpu.VMEM((_NCHUNK * _NR,), jnp.int32),
        pltpu.VMEM((_NR, _D), jnp.float32),
        pltpu.VMEM((_NR, _D), jnp.float32),
        pltpu.VMEM((_CH, _SCORE_COLS), jnp.float32),
        pltpu.VMEM((2 * _CH, 16), jnp.float32),
        pltpu.SemaphoreType.DMA,
        pltpu.SemaphoreType.DMA,
    ],
)


def _loss_body(s_ref, o_ref):
    s = s_ref[...]
    col = lax.broadcasted_iota(jnp.int32, s.shape, 1)
    x = jnp.where(col == 0, s, -s)
    ls = jnp.minimum(x, 0.0) - jnp.log(1.0 + jnp.exp(-jnp.abs(x)))
    ls = jnp.where(col < _NNEG + 1, ls, 0.0)
    o_ref[...] = (-jnp.sum(ls) / _B).reshape(1, 1)


_loss = pl.pallas_call(
    _loss_body,
    out_shape=jax.ShapeDtypeStruct((1, 1), jnp.float32),
)


def kernel(center, context, negatives, input_emb, output_emb):
    idxall = jnp.concatenate(
        [center.reshape(-1, _CH), context.reshape(-1, _CH),
         negatives.reshape(-1, _CH * _NNEG)], axis=1).reshape(-1)
    scores = _sc_scores(idxall, input_emb, output_emb)
    return _loss(scores)[0, 0]


# speedup vs baseline: 4.6413x; 1.0019x over previous
import jax
import jax.numpy as jnp
from jax import lax
from jax.experimental import pallas as pl
from jax.experimental.pallas import tpu as pltpu
from jax.experimental.pallas import tpu_sc as plsc

_V = 1000000
_D = 64
_B = 16384
_NNEG = 20
_NC = 2
_NS = 16
_NW = _NC * _NS
_BPW = _B // _NW
_CH = 16
_NCHUNK = _BPW // _CH
_SCORE_COLS = 32

_NR = _CH * (_NNEG + 2)
_NG = _NR // 16


def _sc_body(idxall_hbm, in_emb_hbm, out_emb_hbm,
             scores_hbm, idx_v, rows_v0, rows_v1, sc_v, pacc,
             sem0, sem1):
    wid = lax.axis_index("s") * _NC + lax.axis_index("c")
    iota = lax.iota(jnp.int32, 16)
    iota_hi = iota + 16
    sems = (sem0, sem1)
    rows_bufs = (rows_v0, rows_v1)

    pltpu.sync_copy(
        idxall_hbm.at[pl.ds(wid * _NCHUNK * _NR, _NCHUNK * _NR)], idx_v)

    def stage(ch, buf, sem):
        rows_v = rows_bufs[buf]

        def issue_c(g, c2):
            v = idx_v[pl.ds(ch * _NR + g * 16, 16)]
            for k in range(16):
                pltpu.async_copy(in_emb_hbm.at[v[k]],
                                 rows_v.at[g * 16 + k], sem)
            return c2

        lax.fori_loop(0, _CH // 16, issue_c, None)

        def issue_o(g, c2):
            v = idx_v[pl.ds(ch * _NR + g * 16, 16)]
            for k in range(16):
                pltpu.async_copy(out_emb_hbm.at[v[k]],
                                 rows_v.at[g * 16 + k], sem)
            return c2

        lax.fori_loop(_CH // 16, _NG, issue_o, None)

    def consume(ch, buf, stage_next):
        rows_v = rows_bufs[buf]
        nxt_rows = rows_bufs[1 - buf]
        nxt_sem = sems[1 - buf]
        off = (ch + 1) * _NR
        have_next = ch + 1 < _NCHUNK

        if stage_next:
            @pl.when(have_next)
            def _():
                v = idx_v[pl.ds(off, 16)]
                for k in range(16):
                    pltpu.async_copy(in_emb_hbm.at[v[k]],
                                     nxt_rows.at[k], nxt_sem)

        pltpu.make_async_copy(out_emb_hbm.at[pl.ds(0, _NR)], rows_v,
                              sems[buf]).wait()

        def b_body(b, carry2):
            if stage_next:
                @pl.when(have_next)
                def _():
                    v = idx_v[pl.ds(off + (1 + b) * 16, 16)]
                    for k in range(16):
                        pltpu.async_copy(out_emb_hbm.at[v[k]],
                                         nxt_rows.at[(1 + b) * 16 + k],
                                         nxt_sem)

                @pl.when(jnp.logical_and(have_next, b < _NG - 1 - _CH))
                def _():
                    v2 = idx_v[pl.ds(off + (1 + _CH + b) * 16, 16)]
                    for k in range(16):
                        pltpu.async_copy(out_emb_hbm.at[v2[k]],
                                         nxt_rows.at[(1 + _CH + b) * 16 + k],
                                         nxt_sem)
            c0 = rows_v[b, pl.ds(0, 16)]
            c1 = rows_v[b, pl.ds(16, 16)]
            c2 = rows_v[b, pl.ds(32, 16)]
            c3 = rows_v[b, pl.ds(48, 16)]
            for j in range(_NNEG + 1):
                if j == 0:
                    row = _CH + b
                else:
                    row = 2 * _CH + b * _NNEG + (j - 1)
                pacc[j, :] = (c0 * rows_v[row, pl.ds(0, 16)]
                              + c1 * rows_v[row, pl.ds(16, 16)]
                              + c2 * rows_v[row, pl.ds(32, 16)]
                              + c3 * rows_v[row, pl.ds(48, 16)])
            s_lo = jnp.zeros((16,), jnp.float32)
            s_hi = jnp.zeros((16,), jnp.float32)
            for l in range(16):
                dcol = (iota + l) & 15
                s_lo = s_lo + plsc.load_gather(pacc, [iota, dcol])
                s_hi = s_hi + plsc.load_gather(pacc, [iota_hi, dcol])
            sc_v[b, pl.ds(0, 16)] = s_lo
            sc_v[b, pl.ds(16, 16)] = s_hi
            return carry2

        lax.fori_loop(0, _CH, b_body, None)
        base = wid * _BPW + ch * _CH
        pltpu.sync_copy(sc_v, scores_hbm.at[pl.ds(base, _CH), :])

    stage(0, 0, sem0)

    def pair_body(i, carry):
        ch = 2 * i
        consume(ch, 0, True)
        consume(ch + 1, 1, True)
        return carry

    lax.fori_loop(0, _NCHUNK // 2, pair_body, None)


_sc_scores = pl.kernel(
    _sc_body,
    out_type=jax.ShapeDtypeStruct((_B, _SCORE_COLS), jnp.float32),
    mesh=plsc.VectorSubcoreMesh(core_axis_name="c", subcore_axis_name="s",
                                num_cores=_NC, num_subcores=_NS),
    compiler_params=pltpu.CompilerParams(needs_layout_passes=False),
    scratch_types=[
        pltpu.VMEM((_NCHUNK * _NR,), jnp.int32),
        pltpu.VMEM((_NR, _D), jnp.float32),
        pltpu.VMEM((_NR, _D), jnp.float32),
        pltpu.VMEM((_CH, _SCORE_COLS), jnp.float32),
        pltpu.VMEM((2 * _CH, 16), jnp.float32),
        pltpu.SemaphoreType.DMA,
        pltpu.SemaphoreType.DMA,
    ],
)


def _loss_body(s_ref, o_ref):
    s = s_ref[...]
    col = lax.broadcasted_iota(jnp.int32, s.shape, 1)
    x = jnp.where(col == 0, s, -s)
    ls = jnp.minimum(x, 0.0) - jnp.log(1.0 + jnp.exp(-jnp.abs(x)))
    ls = jnp.where(col < _NNEG + 1, ls, 0.0)
    o_ref[...] = (-jnp.sum(ls) / _B).reshape(1, 1)


_loss = pl.pallas_call(
    _loss_body,
    out_shape=jax.ShapeDtypeStruct((1, 1), jnp.float32),
)


def kernel(center, context, negatives, input_emb, output_emb):
    idxall = jnp.concatenate(
        [center.reshape(-1, _CH), context.reshape(-1, _CH),
         negatives.reshape(-1, _CH * _NNEG)], axis=1).reshape(-1)
    scores = _sc_scores(idxall, input_emb, output_emb)
    return _loss(scores)[0, 0]
